# baseline (speedup 1.0000x reference)
import jax
import jax.numpy as jnp
import numpy as np
from jax import lax
from jax.experimental import pallas as pl
from jax.experimental.pallas import tpu as pltpu
from jax.experimental.pallas import tpu_sc as plsc

NLVL = 16
TBLSZ = 524288
NFEAT = 2 * NLVL
BTOT = 524288
_growth = np.exp((np.log(512) - np.log(16)) / (NLVL - 1))
RES = [int(np.floor(16 * _growth ** l)) for l in range(NLVL)]
HASH_M = np.int32(np.uint32(2654435761).astype(np.int64) - (1 << 32))
MASK = np.int32(TBLSZ - 1)
HIM = np.int32(~127)
LOM = np.int32(127)

NC, NS, LANES = 2, 16, 16
NW = NC * NS
PPW = BTOT // NW
CH = 128
NCH = PPW // CH

NDENSE = 4
NHASH = NLVL - NDENSE
NIDX = 4 * NHASH * CH
IDXSZ = 2 * NIDX

DH = [RES[l] + 1 for l in range(NDENSE)]
DW = [((RES[l] + 1 + 15) // 16) * 16 for l in range(NDENSE)]
DS = [DH[l] * DW[l] for l in range(NDENSE)]
DOFF = [0] * NDENSE
for _l in range(1, NDENSE):
    DOFF[_l] = DOFF[_l - 1] + 2 * DS[_l - 1]
DENSE_SZ = DOFF[-1] + DS[-1] + IDXSZ


def _encode_body(x_hbm, y_hbm, emb_hbm, f_hbm, xv, yv, idx0, idx1, val0,
                 val1, fv0, fv1, dense, sem0, sem1):
    cid = lax.axis_index("c")
    sid = lax.axis_index("s")
    wid = sid * NC + cid
    wbase = wid * PPW
    pltpu.sync_copy(x_hbm.at[pl.ds(wbase, PPW)], xv)
    pltpu.sync_copy(y_hbm.at[pl.ds(wbase, PPW)], yv)

    iota = lax.iota(jnp.int32, LANES)

    def zero_body(i, c2):
        z = jnp.zeros((LANES,), jnp.int32)
        idx0[pl.ds(i * LANES, LANES)] = z
        idx1[pl.ds(i * LANES, LANES)] = z
        return c2

    lax.fori_loop(0, IDXSZ // LANES, zero_body, 0)

    for l in range(NDENSE):
        W, H, S, off = DW[l], DH[l], DS[l], DOFF[l]
        lofs = np.int32(l * TBLSZ * 2)

        def row_body(gy, c2, W=W, lofs=lofs):
            hyv = jnp.broadcast_to(gy, (LANES,)).astype(jnp.int32) * HASH_M
            for b in range(W // LANES):
                gx = iota + np.int32(b * LANES)
                e = (gx ^ hyv) & MASK
                w0 = (((e & HIM) * np.int32(2)) | (e & LOM)) + lofs
                idx0[pl.ds(gy * W + b * LANES, LANES)] = w0
                idx1[pl.ds(gy * W + b * LANES, LANES)] = w0 + np.int32(128)
            return c2

        lax.fori_loop(0, H, row_body, 0)
        pltpu.async_copy(emb_hbm.at[idx0], dense.at[pl.ds(off, IDXSZ)],
                         sem0).wait()
        pltpu.async_copy(emb_hbm.at[idx1], dense.at[pl.ds(off + S, IDXSZ)],
                         sem1).wait()

    def compute_idx(ci, idxv):
        def idx_body(v, c2):
            p0 = v * LANES
            x = xv[pl.ds(ci * CH + p0, LANES)]
            y = yv[pl.ds(ci * CH + p0, LANES)]
            for li in range(NHASH):
                l = NDENSE + li
                res = np.float32(RES[l])
                px = x * res
                py = y * res
                ix = px.astype(jnp.int32)
                iy = py.astype(jnp.int32)
                hy0 = iy * HASH_M
                hy1 = hy0 + HASH_M
                ix1 = ix + np.int32(1)
                off = np.int32(l * TBLSZ * 2)
                e00 = (ix ^ hy0) & MASK
                e10 = (ix1 ^ hy0) & MASK
                e01 = (ix ^ hy1) & MASK
                e11 = (ix1 ^ hy1) & MASK
                w00 = (((e00 & HIM) * np.int32(2)) | (e00 & LOM)) + off
                w10 = (((e10 & HIM) * np.int32(2)) | (e10 & LOM)) + off
                w01 = (((e01 & HIM) * np.int32(2)) | (e01 & LOM)) + off
                w11 = (((e11 & HIM) * np.int32(2)) | (e11 & LOM)) + off
                lb = li * 4 * CH + p0
                idxv[pl.ds(lb, LANES)] = w00
                idxv[pl.ds(lb + CH, LANES)] = w10
                idxv[pl.ds(lb + 2 * CH, LANES)] = w01
                idxv[pl.ds(lb + 3 * CH, LANES)] = w11
                idxv[pl.ds(NIDX + lb, LANES)] = w00 + np.int32(128)
                idxv[pl.ds(NIDX + lb + CH, LANES)] = w10 + np.int32(128)
                idxv[pl.ds(NIDX + lb + 2 * CH, LANES)] = w01 + np.int32(128)
                idxv[pl.ds(NIDX + lb + 3 * CH, LANES)] = w11 + np.int32(128)
            return c2

        lax.fori_loop(0, CH // LANES, idx_body, 0)

    def interp(ci, vals, featv):
        def interp_body(v, c2):
            p0 = v * LANES
            scat = (p0 + iota) * np.int32(NFEAT)
            x = xv[pl.ds(ci * CH + p0, LANES)]
            y = yv[pl.ds(ci * CH + p0, LANES)]
            for l in range(NLVL):
                res = np.float32(RES[l])
                px = x * res
                py = y * res
                ix = px.astype(jnp.int32)
                iy = py.astype(jnp.int32)
                fx = px - ix.astype(jnp.float32)
                fy = py - iy.astype(jnp.float32)
                gx = np.float32(1.0) - fx
                gy = np.float32(1.0) - fy
                w00 = gx * gy
                w10 = fx * gy
                w01 = gx * fy
                w11 = fx * fy
                if l < NDENSE:
                    W, S, off = DW[l], DS[l], DOFF[l]
                    base = iy * np.int32(W) + ix + np.int32(off)
                    f0 = w00 * plsc.load_gather(dense, [base])
                    f0 = f0 + w10 * plsc.load_gather(dense,
                                                     [base + np.int32(1)])
                    f0 = f0 + w01 * plsc.load_gather(dense,
                                                     [base + np.int32(W)])
                    f0 = f0 + w11 * plsc.load_gather(dense,
                                                     [base + np.int32(W + 1)])
                    b1 = base + np.int32(S)
                    f1 = w00 * plsc.load_gather(dense, [b1])
                    f1 = f1 + w10 * plsc.load_gather(dense,
                                                     [b1 + np.int32(1)])
                    f1 = f1 + w01 * plsc.load_gather(dense,
                                                     [b1 + np.int32(W)])
                    f1 = f1 + w11 * plsc.load_gather(dense,
                                                     [b1 + np.int32(W + 1)])
                else:
                    li = l - NDENSE
                    lb = li * 4 * CH + p0
                    f0 = w00 * vals[pl.ds(lb, LANES)]
                    f0 = f0 + w10 * vals[pl.ds(lb + CH, LANES)]
                    f0 = f0 + w01 * vals[pl.ds(lb + 2 * CH, LANES)]
                    f0 = f0 + w11 * vals[pl.ds(lb + 3 * CH, LANES)]
                    f1 = w00 * vals[pl.ds(NIDX + lb, LANES)]
                    f1 = f1 + w10 * vals[pl.ds(NIDX + lb + CH, LANES)]
                    f1 = f1 + w01 * vals[pl.ds(NIDX + lb + 2 * CH, LANES)]
                    f1 = f1 + w11 * vals[pl.ds(NIDX + lb + 3 * CH, LANES)]
                plsc.store_scatter(featv, [scat + np.int32(2 * l)], f0)
                plsc.store_scatter(featv, [scat + np.int32(2 * l + 1)], f1)
            return c2

        lax.fori_loop(0, CH // LANES, interp_body, 0)
        pltpu.sync_copy(
            featv, f_hbm.at[pl.ds((wbase + ci * CH) * NFEAT, CH * NFEAT)]
        )

    compute_idx(0, idx0)
    pltpu.async_copy(emb_hbm.at[idx0], val0, sem0)

    def pair_body(g, carry):
        c0 = g * 2
        c1 = c0 + 1
        compute_idx(c1, idx1)
        pltpu.async_copy(emb_hbm.at[idx1], val1, sem1)
        pltpu.make_async_copy(emb_hbm.at[idx0], val0, sem0).wait()
        interp(c0, val0, fv0)

        @pl.when(c1 + 1 < NCH)
        def _():
            compute_idx(c1 + 1, idx0)
            pltpu.async_copy(emb_hbm.at[idx0], val0, sem0)

        pltpu.make_async_copy(emb_hbm.at[idx1], val1, sem1).wait()
        interp(c1, val1, fv1)
        return carry

    lax.fori_loop(0, NCH // 2, pair_body, 0)


_encode = pl.kernel(
    _encode_body,
    out_type=jax.ShapeDtypeStruct((BTOT * NFEAT,), jnp.float32),
    mesh=plsc.VectorSubcoreMesh(
        core_axis_name="c", subcore_axis_name="s", num_cores=NC, num_subcores=NS
    ),
    compiler_params=pltpu.CompilerParams(needs_layout_passes=False),
    scratch_types=[
        pltpu.VMEM((PPW,), jnp.float32),
        pltpu.VMEM((PPW,), jnp.float32),
        pltpu.VMEM((IDXSZ,), jnp.int32),
        pltpu.VMEM((IDXSZ,), jnp.int32),
        pltpu.VMEM((IDXSZ,), jnp.float32),
        pltpu.VMEM((IDXSZ,), jnp.float32),
        pltpu.VMEM((CH * NFEAT,), jnp.float32),
        pltpu.VMEM((CH * NFEAT,), jnp.float32),
        pltpu.VMEM((DENSE_SZ,), jnp.float32),
        pltpu.SemaphoreType.DMA,
        pltpu.SemaphoreType.DMA,
    ],
)

MLP_TB = 8192


def _mlp_body(f_ref, w1_ref, b1_ref, w2_ref, b2_ref, w3_ref, b3_ref, o_ref):
    ft = f_ref[...]
    h = lax.dot_general(ft, w1_ref[...], (((1,), (0,)), ((), ())),
                        preferred_element_type=jnp.float32)
    h = jnp.maximum(h + b1_ref[...], 0.0)
    h = lax.dot_general(h, w2_ref[...], (((1,), (0,)), ((), ())),
                        preferred_element_type=jnp.float32)
    h = jnp.maximum(h + b2_ref[...], 0.0)
    h = lax.dot_general(h, w3_ref[...], (((1,), (0,)), ((), ())),
                        preferred_element_type=jnp.float32)
    h = h + b3_ref[...]
    o_ref[...] = 1.0 / (1.0 + jnp.exp(-h))


def _mlp(f, W1, b1, W2, b2, W3, b3):
    return pl.pallas_call(
        _mlp_body,
        grid=(BTOT // MLP_TB,),
        in_specs=[
            pl.BlockSpec((MLP_TB, NFEAT), lambda i: (i, 0)),
            pl.BlockSpec((NFEAT, 64), lambda i: (0, 0)),
            pl.BlockSpec((1, 64), lambda i: (0, 0)),
            pl.BlockSpec((64, 64), lambda i: (0, 0)),
            pl.BlockSpec((1, 64), lambda i: (0, 0)),
            pl.BlockSpec((64, 3), lambda i: (0, 0)),
            pl.BlockSpec((1, 3), lambda i: (0, 0)),
        ],
        out_specs=pl.BlockSpec((MLP_TB, 3), lambda i: (i, 0)),
        out_shape=jax.ShapeDtypeStruct((BTOT, 3), jnp.float32),
    )(f, W1, b1, W2, b2, W3, b3)


def kernel(xn, embeddings, W1, b1, W2, b2, W3, b3):
    x = xn[:, 0]
    y = xn[:, 1]
    emb_flat = (
        embeddings.reshape(NLVL, TBLSZ // 128, 128, 2)
        .transpose(0, 1, 3, 2)
        .reshape(-1)
    )
    f = _encode(x, y, emb_flat).reshape(BTOT, NFEAT)
    return _mlp(f, W1, b1.reshape(1, -1), W2, b2.reshape(1, -1),
                W3, b3.reshape(1, -1))


# speedup vs baseline: 4.5582x; 4.5582x over previous
import jax
import jax.numpy as jnp
import numpy as np
from jax import lax
from jax.experimental import pallas as pl
from jax.experimental.pallas import tpu as pltpu
from jax.experimental.pallas import tpu_sc as plsc

NLVL = 16
TBLSZ = 524288
NFEAT = 2 * NLVL
BTOT = 524288
_growth = np.exp((np.log(512) - np.log(16)) / (NLVL - 1))
RES = [int(np.floor(16 * _growth ** l)) for l in range(NLVL)]
HASH_M = np.int32(np.uint32(2654435761).astype(np.int64) - (1 << 32))
MASK = np.int32(TBLSZ - 1)

NC, NS, LANES = 2, 16, 16
NW = NC * NS
PPW = BTOT // NW
CH = 128
NCH = PPW // CH
NIDX = 4 * NLVL * CH


def _encode_body(x_hbm, y_hbm, emb_hbm, f_hbm, xv, yv, idx0, idx1, val0,
                 val1, fv0, fv1, sem0, sem1):
    cid = lax.axis_index("c")
    sid = lax.axis_index("s")
    wid = sid * NC + cid
    wbase = wid * PPW
    pltpu.sync_copy(x_hbm.at[pl.ds(wbase, PPW)], xv)
    pltpu.sync_copy(y_hbm.at[pl.ds(wbase, PPW)], yv)

    def compute_idx(ci, idxv):
        def idx_body(v, c2):
            p0 = v * LANES
            x = xv[pl.ds(ci * CH + p0, LANES)]
            y = yv[pl.ds(ci * CH + p0, LANES)]
            for l in range(NLVL):
                res = np.float32(RES[l])
                px = x * res
                py = y * res
                ix = px.astype(jnp.int32)
                iy = py.astype(jnp.int32)
                hy0 = iy * HASH_M
                hy1 = hy0 + HASH_M
                ix1 = ix + np.int32(1)
                off = np.int32(l * TBLSZ * 2)
                e00 = (ix ^ hy0) & MASK
                e10 = (ix1 ^ hy0) & MASK
                e01 = (ix ^ hy1) & MASK
                e11 = (ix1 ^ hy1) & MASK
                HIM = np.int32(~127)
                LOM = np.int32(127)
                w00 = (((e00 & HIM) * np.int32(2)) | (e00 & LOM)) + off
                w10 = (((e10 & HIM) * np.int32(2)) | (e10 & LOM)) + off
                w01 = (((e01 & HIM) * np.int32(2)) | (e01 & LOM)) + off
                w11 = (((e11 & HIM) * np.int32(2)) | (e11 & LOM)) + off
                lb = l * 4 * CH + p0
                idxv[pl.ds(lb, LANES)] = w00
                idxv[pl.ds(lb + CH, LANES)] = w10
                idxv[pl.ds(lb + 2 * CH, LANES)] = w01
                idxv[pl.ds(lb + 3 * CH, LANES)] = w11
                idxv[pl.ds(NIDX + lb, LANES)] = w00 + np.int32(128)
                idxv[pl.ds(NIDX + lb + CH, LANES)] = w10 + np.int32(128)
                idxv[pl.ds(NIDX + lb + 2 * CH, LANES)] = w01 + np.int32(128)
                idxv[pl.ds(NIDX + lb + 3 * CH, LANES)] = w11 + np.int32(128)
            return c2

        lax.fori_loop(0, CH // LANES, idx_body, 0)

    def interp(ci, vals, featv):
        def interp_body(v, c2):
            p0 = v * LANES
            x = xv[pl.ds(ci * CH + p0, LANES)]
            y = yv[pl.ds(ci * CH + p0, LANES)]
            for l in range(NLVL):
                res = np.float32(RES[l])
                px = x * res
                py = y * res
                ix = px.astype(jnp.int32)
                iy = py.astype(jnp.int32)
                fx = px - ix.astype(jnp.float32)
                fy = py - iy.astype(jnp.float32)
                gx = np.float32(1.0) - fx
                gy = np.float32(1.0) - fy
                w00 = gx * gy
                w10 = fx * gy
                w01 = gx * fy
                w11 = fx * fy
                lb = l * 4 * CH + p0
                f0 = w00 * vals[pl.ds(lb, LANES)]
                f0 = f0 + w10 * vals[pl.ds(lb + CH, LANES)]
                f0 = f0 + w01 * vals[pl.ds(lb + 2 * CH, LANES)]
                f0 = f0 + w11 * vals[pl.ds(lb + 3 * CH, LANES)]
                f1 = w00 * vals[pl.ds(NIDX + lb, LANES)]
                f1 = f1 + w10 * vals[pl.ds(NIDX + lb + CH, LANES)]
                f1 = f1 + w01 * vals[pl.ds(NIDX + lb + 2 * CH, LANES)]
                f1 = f1 + w11 * vals[pl.ds(NIDX + lb + 3 * CH, LANES)]
                featv[pl.ds(2 * l * CH + p0, LANES)] = f0
                featv[pl.ds((2 * l + 1) * CH + p0, LANES)] = f1
            return c2

        lax.fori_loop(0, CH // LANES, interp_body, 0)
        pltpu.sync_copy(
            featv, f_hbm.at[pl.ds((wbase + ci * CH) * NFEAT, CH * NFEAT)]
        )

    compute_idx(0, idx0)
    pltpu.async_copy(emb_hbm.at[idx0], val0, sem0)

    def pair_body(g, carry):
        c0 = g * 2
        c1 = c0 + 1
        compute_idx(c1, idx1)
        pltpu.async_copy(emb_hbm.at[idx1], val1, sem1)
        pltpu.make_async_copy(emb_hbm.at[idx0], val0, sem0).wait()
        interp(c0, val0, fv0)

        @pl.when(c1 + 1 < NCH)
        def _():
            compute_idx(c1 + 1, idx0)
            pltpu.async_copy(emb_hbm.at[idx0], val0, sem0)

        pltpu.make_async_copy(emb_hbm.at[idx1], val1, sem1).wait()
        interp(c1, val1, fv1)
        return carry

    lax.fori_loop(0, NCH // 2, pair_body, 0)


_encode = pl.kernel(
    _encode_body,
    out_type=jax.ShapeDtypeStruct((BTOT * NFEAT,), jnp.float32),
    mesh=plsc.VectorSubcoreMesh(
        core_axis_name="c", subcore_axis_name="s", num_cores=NC, num_subcores=NS
    ),
    compiler_params=pltpu.CompilerParams(needs_layout_passes=False),
    scratch_types=[
        pltpu.VMEM((PPW,), jnp.float32),
        pltpu.VMEM((PPW,), jnp.float32),
        pltpu.VMEM((2 * NIDX,), jnp.int32),
        pltpu.VMEM((2 * NIDX,), jnp.int32),
        pltpu.VMEM((2 * NIDX,), jnp.float32),
        pltpu.VMEM((2 * NIDX,), jnp.float32),
        pltpu.VMEM((CH * NFEAT,), jnp.float32),
        pltpu.VMEM((CH * NFEAT,), jnp.float32),
        pltpu.SemaphoreType.DMA,
        pltpu.SemaphoreType.DMA,
    ],
)

MLP_TB = 8192


def _mlp_body(f_ref, w1_ref, b1_ref, w2_ref, b2_ref, w3_ref, b3_ref, o_ref):
    ft = f_ref[...]
    h = lax.dot_general(ft, w1_ref[...], (((1,), (0,)), ((), ())),
                        preferred_element_type=jnp.float32)
    h = h.reshape(MLP_TB, 64)
    h = jnp.maximum(h + b1_ref[...], 0.0)
    h = lax.dot_general(h, w2_ref[...], (((1,), (0,)), ((), ())),
                        preferred_element_type=jnp.float32)
    h = jnp.maximum(h + b2_ref[...], 0.0)
    h = lax.dot_general(h, w3_ref[...], (((1,), (0,)), ((), ())),
                        preferred_element_type=jnp.float32)
    h = h + b3_ref[...]
    o_ref[...] = 1.0 / (1.0 + jnp.exp(-h))


def _mlp(f, W1, b1, W2, b2, W3, b3):
    return pl.pallas_call(
        _mlp_body,
        grid=(BTOT // MLP_TB,),
        in_specs=[
            pl.BlockSpec((MLP_TB // CH, NFEAT, CH), lambda i: (i, 0, 0)),
            pl.BlockSpec((NFEAT, 64), lambda i: (0, 0)),
            pl.BlockSpec((1, 64), lambda i: (0, 0)),
            pl.BlockSpec((64, 64), lambda i: (0, 0)),
            pl.BlockSpec((1, 64), lambda i: (0, 0)),
            pl.BlockSpec((64, 3), lambda i: (0, 0)),
            pl.BlockSpec((1, 3), lambda i: (0, 0)),
        ],
        out_specs=pl.BlockSpec((MLP_TB, 3), lambda i: (i, 0)),
        out_shape=jax.ShapeDtypeStruct((BTOT, 3), jnp.float32),
    )(f, W1, b1, W2, b2, W3, b3)


def kernel(xn, embeddings, W1, b1, W2, b2, W3, b3):
    x = xn[:, 0]
    y = xn[:, 1]
    emb_flat = (
        embeddings.reshape(NLVL, TBLSZ // 128, 128, 2)
        .transpose(0, 1, 3, 2)
        .reshape(-1)
    )
    f = _encode(x, y, emb_flat).reshape(BTOT // CH, NFEAT, CH)
    return _mlp(f, W1, b1.reshape(1, -1), W2, b2.reshape(1, -1),
                W3, b3.reshape(1, -1))
